# Initial kernel scaffold; baseline (speedup 1.0000x reference)
import jax
import jax.numpy as jnp
from jax import lax
from jax.experimental import pallas as pl
from jax.experimental.pallas import tpu as pltpu

_EPS = 1e-5


def _enc_vq_body(mels, w1, w2, w3, w4, w5, w6, b6,
                 g1, be1, g2, be2, g3, be3, g4, be4, g5, be5,
                 cb,
                 q_out, loss_out, perp_out,
                 s1, s2, s3, s4, s5):
    f32 = jnp.float32

    def conv_batch(src_ref, w_ref, b, K, t_out, t_src_off):
        acc = None
        for k in range(K):
            wk = w_ref[:, :, k]
            xk = src_ref[b, :, t_src_off + k:t_src_off + k + t_out]
            p = jnp.dot(wk, xk, preferred_element_type=f32)
            acc = p if acc is None else acc + p
        return acc

    def bn_relu_inplace(s_ref, lo, hi, g_ref, be_ref):
        n = 4 * (hi - lo)
        tot = s_ref[0, :, lo:hi] + s_ref[1, :, lo:hi] \
            + s_ref[2, :, lo:hi] + s_ref[3, :, lo:hi]
        mean = jnp.sum(tot, axis=1, keepdims=True) / n
        sq = (s_ref[0, :, lo:hi] ** 2 + s_ref[1, :, lo:hi] ** 2
              + s_ref[2, :, lo:hi] ** 2 + s_ref[3, :, lo:hi] ** 2)
        var = jnp.sum(sq, axis=1, keepdims=True) / n - mean ** 2
        scale = g_ref[...] / jnp.sqrt(var + _EPS)
        shift = be_ref[...] - scale * mean
        for b in range(4):
            y = s_ref[b, :, lo:hi]
            s_ref[b, :, lo:hi] = jnp.maximum(y * scale + shift, 0.0)

    zc512 = jnp.zeros((512, 1), f32)

    for b in range(4):
        y = conv_batch(mels, w1, b, 3, 64, 0)
        s1[b, :, 1:65] = y
        s1[b, :, 0:1] = zc512
        s1[b, :, 65:66] = zc512
    bn_relu_inplace(s1, 1, 65, g1, be1)

    for b in range(4):
        y = conv_batch(s1, w2, b, 3, 64, 0)
        s2[b, :, 1:65] = y
        s2[b, :, 0:1] = zc512
        s2[b, :, 65:66] = zc512
    bn_relu_inplace(s2, 1, 65, g2, be2)

    r_i = lax.broadcasted_iota(jnp.int32, (63, 32), 0)
    c_i = lax.broadcasted_iota(jnp.int32, (63, 32), 1)
    sel = (r_i == 2 * c_i).astype(f32)
    for b in range(4):
        z = conv_batch(s2, w3, b, 4, 63, 0)
        y = jnp.dot(z, sel, preferred_element_type=f32)
        s3[b, :, 1:33] = y
        s3[b, :, 0:1] = zc512
        s3[b, :, 33:34] = zc512
    bn_relu_inplace(s3, 1, 33, g3, be3)

    for b in range(4):
        y = conv_batch(s3, w4, b, 3, 32, 0)
        s4[b, :, 1:33] = y
        s4[b, :, 0:1] = zc512
        s4[b, :, 33:34] = zc512
    bn_relu_inplace(s4, 1, 33, g4, be4)

    for b in range(4):
        y = conv_batch(s4, w5, b, 3, 32, 0)
        s5[b, :, 0:32] = y
    bn_relu_inplace(s5, 0, 32, g5, be5)

    e2 = jnp.sum(cb[...] ** 2, axis=1, keepdims=True)
    iota_codes_r = lax.broadcasted_iota(jnp.int32, (32, 512), 1)
    iota_codes_c = lax.broadcasted_iota(jnp.int32, (512, 32), 0)
    big = jnp.full((512, 32), 1 << 20, jnp.int32)

    counts = jnp.zeros((512, 1), f32)
    z2_sum = 0.0
    zq_sum = 0.0
    q2_sum = 0.0
    for b in range(4):
        h6 = jnp.dot(w6[...], s5[b, :, 0:32],
                     preferred_element_type=f32) + b6[...]
        s = jnp.dot(cb[...], h6, preferred_element_type=f32)
        z2 = jnp.sum(h6 ** 2, axis=0, keepdims=True)
        dist = e2 + z2 - 2.0 * s
        mn = jnp.min(dist, axis=0, keepdims=True)
        cand = jnp.where(dist == mn, iota_codes_c, big)
        idx = jnp.min(cand, axis=0, keepdims=True)
        oneh_jt = (iota_codes_c == idx).astype(f32)
        oneh_tj = (iota_codes_r == idx.reshape(32, 1)).astype(f32)
        q_b = jnp.dot(oneh_tj, cb[...], preferred_element_type=f32)
        q_out[b * 32:(b + 1) * 32, :] = q_b
        counts = counts + jnp.sum(oneh_jt, axis=1, keepdims=True)
        z2_sum = z2_sum + jnp.sum(z2)
        zq_sum = zq_sum + jnp.sum(oneh_jt * s)
        q2_sum = q2_sum + jnp.sum(oneh_jt * e2)

    e_latent = (z2_sum - 2.0 * zq_sum + q2_sum) / (128.0 * 64.0)
    loss_out[0, 0] = 0.25 * e_latent
    p = counts / 128.0
    perp_out[0, 0] = jnp.exp(-jnp.sum(p * jnp.log(p + 1e-10)))


def _run_enc_vq(mels, p):
    f32 = jnp.float32
    outs = (
        jax.ShapeDtypeStruct((128, 64), f32),
        jax.ShapeDtypeStruct((1, 1), f32),
        jax.ShapeDtypeStruct((1, 1), f32),
    )
    scratch = [
        pltpu.VMEM((4, 512, 66), f32),
        pltpu.VMEM((4, 512, 66), f32),
        pltpu.VMEM((4, 512, 34), f32),
        pltpu.VMEM((4, 512, 34), f32),
        pltpu.VMEM((4, 512, 32), f32),
    ]
    args = (
        mels,
        p['enc_w1'], p['enc_w2'], p['enc_w3'], p['enc_w4'], p['enc_w5'],
        p['enc_w6'][:, :, 0], p['enc_b6'].reshape(64, 1),
        p['bn1_g'].reshape(512, 1), p['bn1_b'].reshape(512, 1),
        p['bn2_g'].reshape(512, 1), p['bn2_b'].reshape(512, 1),
        p['bn3_g'].reshape(512, 1), p['bn3_b'].reshape(512, 1),
        p['bn4_g'].reshape(512, 1), p['bn4_b'].reshape(512, 1),
        p['bn5_g'].reshape(512, 1), p['bn5_b'].reshape(512, 1),
        p['codebook'],
    )
    return pl.pallas_call(
        _enc_vq_body,
        out_shape=outs,
        scratch_shapes=scratch,
    )(*args)


def _rnn1_body(m0,
               wi0f, wh0f, bi0f, bh0f, wi0r, wh0r, bi0r, bh0r,
               wi1f, wh1f, bi1f, bh1f, wi1r, wh1r, bi1r, bh1r,
               out, y0):
    H = 128

    def gru(in_ref, wi, wh, bi, bh, out_ref, off, reverse):
        def step(i, h):
            t = 63 - i if reverse else i
            xt = in_ref[t]
            gi = jnp.dot(xt, wi[...], preferred_element_type=jnp.float32) \
                + bi[...]
            gh = jnp.dot(h, wh[...], preferred_element_type=jnp.float32) \
                + bh[...]
            r = jax.nn.sigmoid(gi[:, 0:H] + gh[:, 0:H])
            z = jax.nn.sigmoid(gi[:, H:2 * H] + gh[:, H:2 * H])
            n = jnp.tanh(gi[:, 2 * H:3 * H] + r * gh[:, 2 * H:3 * H])
            hn = (1.0 - z) * n + z * h
            out_ref[t, :, off:off + H] = hn
            return hn
        lax.fori_loop(0, 64, step, jnp.zeros((4, H), jnp.float32))

    gru(m0, wi0f, wh0f, bi0f, bh0f, y0, 0, False)
    gru(m0, wi0r, wh0r, bi0r, bh0r, y0, H, True)
    gru(y0, wi1f, wh1f, bi1f, bh1f, out, 0, False)
    gru(y0, wi1r, wh1r, bi1r, bh1r, out, H, True)


def _run_rnn1(m0, p):
    f32 = jnp.float32
    args = [m0]
    for l in range(2):
        for d in ['f', 'r']:
            args.append(p['rnn1_l%d_%s_w_ih' % (l, d)].T)
            args.append(p['rnn1_l%d_%s_w_hh' % (l, d)].T)
            args.append(p['rnn1_l%d_%s_b_ih' % (l, d)].reshape(1, 384))
            args.append(p['rnn1_l%d_%s_b_hh' % (l, d)].reshape(1, 384))
    return pl.pallas_call(
        _rnn1_body,
        out_shape=jax.ShapeDtypeStruct((64, 4, 256), f32),
        scratch_shapes=[pltpu.VMEM((64, 4, 256), f32)],
    )(*args)


def _rnn2_body(xs, aemb, wxt, wmt, bih, mrows, whht, bhh,
               fc1t, fc1b, fc2t, fc2b,
               out, U_s, V_s, G_s, H_s, h_s):
    f32 = jnp.float32
    c = pl.program_id(0)

    @pl.when(c == 0)
    def _init():
        U_s[...] = jnp.dot(aemb[...], wxt[...], preferred_element_type=f32)
        V_s[...] = jnp.dot(mrows[...], wmt[...],
                           preferred_element_type=f32) + bih[...]
        h_s[...] = jnp.zeros((4, 512), f32)

    x_blk = xs[0]
    xf = x_blk.reshape(256, 1)
    iota_v = lax.broadcasted_iota(jnp.int32, (256, 256), 1)
    oneh = (xf == iota_v).astype(f32)
    G = jnp.dot(oneh, U_s[...], preferred_element_type=f32)
    Vc = V_s[pl.ds(c * 4, 4), :]
    G_s[...] = G.reshape(64, 4, 1536) + Vc[None, :, :]

    R = 512

    def step(s, h):
        gi = G_s[s]
        gh = jnp.dot(h, whht[...], preferred_element_type=f32) + bhh[...]
        r = jax.nn.sigmoid(gi[:, 0:R] + gh[:, 0:R])
        z = jax.nn.sigmoid(gi[:, R:2 * R] + gh[:, R:2 * R])
        n = jnp.tanh(gi[:, 2 * R:3 * R] + r * gh[:, 2 * R:3 * R])
        hn = (1.0 - z) * n + z * h
        H_s[s] = hn
        return hn

    h = lax.fori_loop(0, 64, step, h_s[...])
    h_s[...] = h

    Hm = H_s[...].reshape(256, 512)
    o = jnp.maximum(
        jnp.dot(Hm, fc1t[...], preferred_element_type=f32) + fc1b[...], 0.0)
    lg = jnp.dot(o, fc2t[...], preferred_element_type=f32) + fc2b[...]
    out[0] = lg.reshape(64, 4, 256)


def _run_rnn2(x, mrows, p):
    f32 = jnp.float32
    xs3 = x.T.reshape(64, 64, 4)
    wih = p['rnn2_w_ih']
    args = (
        xs3,
        p['audio_emb'],
        wih[:, :256].T, wih[:, 256:].T,
        p['rnn2_b_ih'].reshape(1, 1536),
        mrows,
        p['rnn2_w_hh'].T,
        p['rnn2_b_hh'].reshape(1, 1536),
        p['fc1_w'].T, p['fc1_b'].reshape(1, 256),
        p['fc2_w'].T, p['fc2_b'].reshape(1, 256),
    )
    const = lambda shape: pl.BlockSpec(shape, lambda c: (0,) * len(shape))
    in_specs = [
        pl.BlockSpec((1, 64, 4), lambda c: (c, 0, 0)),
        const((256, 256)),
        const((256, 1536)), const((256, 1536)),
        const((1, 1536)),
        const((256, 256)),
        const((512, 1536)),
        const((1, 1536)),
        const((512, 256)), const((1, 256)),
        const((256, 256)), const((1, 256)),
    ]
    out = pl.pallas_call(
        _rnn2_body,
        grid=(64,),
        in_specs=in_specs,
        out_specs=pl.BlockSpec((1, 64, 4, 256), lambda c: (c, 0, 0, 0)),
        out_shape=jax.ShapeDtypeStruct((64, 64, 4, 256), f32),
        scratch_shapes=[
            pltpu.VMEM((256, 1536), f32),
            pltpu.VMEM((256, 1536), f32),
            pltpu.VMEM((64, 4, 1536), f32),
            pltpu.VMEM((64, 4, 512), f32),
            pltpu.VMEM((4, 512), f32),
        ],
    )(*args)
    return out.reshape(4096, 4, 256).transpose(1, 0, 2)


def kernel(x, mels, speakers, params):
    p = params
    q, loss, perp = _run_enc_vq(mels, p)

    q3 = q.reshape(4, 32, 64)
    qrep = jnp.repeat(q3, 2, axis=1)
    spk = p['spk_emb'][speakers]
    spk_b = jnp.broadcast_to(spk[:, None, :], (4, 64, 64))
    m0 = jnp.concatenate([qrep, spk_b], axis=-1)
    m0 = m0.transpose(1, 0, 2)

    y1 = _run_rnn1(m0, p)
    mrows = y1.reshape(256, 256)

    logits = _run_rnn2(x, mrows, p)
    return logits, loss[0, 0], perp[0, 0]


# speedup vs baseline: 5.9626x; 5.9626x over previous
import jax
import jax.numpy as jnp
from jax import lax
from jax.experimental import pallas as pl
from jax.experimental.pallas import tpu as pltpu

_EPS = 1e-5


def _shift_mat(t_in, t_out, off):
    u = lax.broadcasted_iota(jnp.int32, (t_in, t_out), 0)
    t = lax.broadcasted_iota(jnp.int32, (t_in, t_out), 1)
    return (u == t + off).astype(jnp.float32)


def _shift_mat_s2(t_in, t_out, off):
    u = lax.broadcasted_iota(jnp.int32, (t_in, t_out), 0)
    t = lax.broadcasted_iota(jnp.int32, (t_in, t_out), 1)
    return (u == 2 * t + off).astype(jnp.float32)


def _enc_vq_body(mels, w1, w2, w3, w4, w5, w6, b6,
                 g1, be1, g2, be2, g3, be3, g4, be4, g5, be5,
                 cb, cbt,
                 q_out, loss_out, perp_out,
                 sa, sb, sc, sd):
    f32 = jnp.float32

    def conv(src_ref, w_ref, b, sels):
        acc = None
        for k, S in enumerate(sels):
            m = jnp.dot(w_ref[k], src_ref[b], preferred_element_type=f32)
            p = m if S is None else jnp.dot(m, S, preferred_element_type=f32)
            acc = p if acc is None else acc + p
        return acc

    def bn_relu_inplace(s_ref, n_t, g_ref, be_ref):
        n = 4 * n_t
        tot = s_ref[0] + s_ref[1] + s_ref[2] + s_ref[3]
        mean = jnp.sum(tot, axis=1, keepdims=True) / n
        sq = s_ref[0] ** 2 + s_ref[1] ** 2 + s_ref[2] ** 2 + s_ref[3] ** 2
        var = jnp.sum(sq, axis=1, keepdims=True) / n - mean ** 2
        scale = g_ref[...] / jnp.sqrt(var + _EPS)
        shift = be_ref[...] - scale * mean
        for b in range(4):
            s_ref[b] = jnp.maximum(s_ref[b] * scale + shift, 0.0)

    sels1 = [_shift_mat(66, 64, k) for k in range(3)]
    for b in range(4):
        sa[b] = conv(mels, w1, b, sels1)
    bn_relu_inplace(sa, 64, g1, be1)

    sels2 = [_shift_mat(64, 64, -1), None, _shift_mat(64, 64, 1)]
    for b in range(4):
        sb[b] = conv(sa, w2, b, sels2)
    bn_relu_inplace(sb, 64, g2, be2)

    sels3 = [_shift_mat_s2(64, 32, k - 1) for k in range(4)]
    for b in range(4):
        sc[b] = conv(sb, w3, b, sels3)
    bn_relu_inplace(sc, 32, g3, be3)

    sels4 = [_shift_mat(32, 32, -1), None, _shift_mat(32, 32, 1)]
    for b in range(4):
        sd[b] = conv(sc, w4, b, sels4)
    bn_relu_inplace(sd, 32, g4, be4)

    for b in range(4):
        sc[b] = conv(sd, w5, b, sels4)
    bn_relu_inplace(sc, 32, g5, be5)

    e2 = jnp.sum(cb[...] ** 2, axis=1, keepdims=True)
    iota_codes = lax.broadcasted_iota(jnp.int32, (512, 32), 0)
    big = jnp.full((512, 32), 1 << 20, jnp.int32)

    counts = jnp.zeros((512, 1), f32)
    z2_sum = 0.0
    zq_sum = 0.0
    q2_sum = 0.0
    for b in range(4):
        h6 = jnp.dot(w6[...], sc[b],
                     preferred_element_type=f32) + b6[...]
        s = jnp.dot(cb[...], h6, preferred_element_type=f32)
        z2 = jnp.sum(h6 ** 2, axis=0, keepdims=True)
        dist = e2 + z2 - 2.0 * s
        mn = jnp.min(dist, axis=0, keepdims=True)
        cand = jnp.where(dist == mn, iota_codes, big)
        idx = jnp.min(cand, axis=0, keepdims=True)
        oneh = (iota_codes == idx).astype(f32)
        q_out[b] = jnp.dot(cbt[...], oneh, preferred_element_type=f32)
        counts = counts + jnp.sum(oneh, axis=1, keepdims=True)
        z2_sum = z2_sum + jnp.sum(z2)
        zq_sum = zq_sum + jnp.sum(oneh * s)
        q2_sum = q2_sum + jnp.sum(oneh * e2)

    e_latent = (z2_sum - 2.0 * zq_sum + q2_sum) / (128.0 * 64.0)
    loss_out[...] = jnp.full((1, 1), 0.25 * e_latent, f32)
    p = counts / 128.0
    perp_out[...] = jnp.full(
        (1, 1), jnp.exp(-jnp.sum(p * jnp.log(p + 1e-10))), f32)


def _run_enc_vq(mels, p):
    f32 = jnp.float32
    outs = (
        jax.ShapeDtypeStruct((4, 64, 32), f32),
        jax.ShapeDtypeStruct((1, 1), f32),
        jax.ShapeDtypeStruct((1, 1), f32),
    )
    scratch = [
        pltpu.VMEM((4, 512, 64), f32),
        pltpu.VMEM((4, 512, 64), f32),
        pltpu.VMEM((4, 512, 32), f32),
        pltpu.VMEM((4, 512, 32), f32),
    ]
    taps = lambda w: jnp.moveaxis(w, 2, 0)
    args = (
        mels,
        taps(p['enc_w1']), taps(p['enc_w2']), taps(p['enc_w3']),
        taps(p['enc_w4']), taps(p['enc_w5']),
        p['enc_w6'][:, :, 0], p['enc_b6'].reshape(64, 1),
        p['bn1_g'].reshape(512, 1), p['bn1_b'].reshape(512, 1),
        p['bn2_g'].reshape(512, 1), p['bn2_b'].reshape(512, 1),
        p['bn3_g'].reshape(512, 1), p['bn3_b'].reshape(512, 1),
        p['bn4_g'].reshape(512, 1), p['bn4_b'].reshape(512, 1),
        p['bn5_g'].reshape(512, 1), p['bn5_b'].reshape(512, 1),
        p['codebook'], p['codebook'].T,
    )
    return pl.pallas_call(
        _enc_vq_body,
        out_shape=outs,
        scratch_shapes=scratch,
    )(*args)


def _rnn1_body(m0,
               wi0f, wh0f, bi0f, bh0f, wi0r, wh0r, bi0r, bh0r,
               wi1f, wh1f, bi1f, bh1f, wi1r, wh1r, bi1r, bh1r,
               out, y0):
    H = 128

    def gru(in_ref, wi, wh, bi, bh, out_ref, off, reverse):
        def step(i, h):
            t = 63 - i if reverse else i
            xt = in_ref[t]
            gi = jnp.dot(xt, wi[...], preferred_element_type=jnp.float32) \
                + bi[...]
            gh = jnp.dot(h, wh[...], preferred_element_type=jnp.float32) \
                + bh[...]
            r = jax.nn.sigmoid(gi[:, 0:H] + gh[:, 0:H])
            z = jax.nn.sigmoid(gi[:, H:2 * H] + gh[:, H:2 * H])
            n = jnp.tanh(gi[:, 2 * H:3 * H] + r * gh[:, 2 * H:3 * H])
            hn = (1.0 - z) * n + z * h
            out_ref[t, :, off:off + H] = hn
            return hn
        lax.fori_loop(0, 64, step, jnp.zeros((4, H), jnp.float32))

    gru(m0, wi0f, wh0f, bi0f, bh0f, y0, 0, False)
    gru(m0, wi0r, wh0r, bi0r, bh0r, y0, H, True)
    gru(y0, wi1f, wh1f, bi1f, bh1f, out, 0, False)
    gru(y0, wi1r, wh1r, bi1r, bh1r, out, H, True)


def _run_rnn1(m0, p):
    f32 = jnp.float32
    args = [m0]
    for l in range(2):
        for d in ['f', 'r']:
            args.append(p['rnn1_l%d_%s_w_ih' % (l, d)].T)
            args.append(p['rnn1_l%d_%s_w_hh' % (l, d)].T)
            args.append(p['rnn1_l%d_%s_b_ih' % (l, d)].reshape(1, 384))
            args.append(p['rnn1_l%d_%s_b_hh' % (l, d)].reshape(1, 384))
    return pl.pallas_call(
        _rnn1_body,
        out_shape=jax.ShapeDtypeStruct((64, 4, 256), f32),
        scratch_shapes=[pltpu.VMEM((64, 4, 256), f32)],
    )(*args)


def _rnn2_body(xs, aemb, wxt, wmt, bih, mrows, whht, bhh,
               fc1t, fc1b, fc2t, fc2b,
               out, U_s, V_s, G_s, H_s, h_s):
    f32 = jnp.float32
    c = pl.program_id(0)

    @pl.when(c == 0)
    def _init():
        U_s[...] = jnp.dot(aemb[...], wxt[...], preferred_element_type=f32)
        V_s[...] = jnp.dot(mrows[...], wmt[...],
                           preferred_element_type=f32) + bih[...]
        h_s[...] = jnp.zeros((4, 512), f32)

    @pl.when(c == 0)
    def _init2():
        H_s[...] = jnp.zeros((512, 512), f32)

    x_blk = xs[0]
    iota_v = lax.broadcasted_iota(jnp.int32, (512, 256), 1)
    oneh = (x_blk == iota_v).astype(f32)
    G_s[...] = jnp.dot(oneh, U_s[...], preferred_element_type=f32)
    Vc = V_s[pl.ds(pl.multiple_of(c * 8, 8), 4), :]

    R = 512

    def step(s, h):
        o8 = pl.multiple_of(s * 8, 8)
        gi = G_s[pl.ds(o8, 4), :] + Vc
        gh = jnp.dot(h, whht[...], preferred_element_type=f32) + bhh[...]
        r = jax.nn.sigmoid(gi[:, 0:R] + gh[:, 0:R])
        z = jax.nn.sigmoid(gi[:, R:2 * R] + gh[:, R:2 * R])
        n = jnp.tanh(gi[:, 2 * R:3 * R] + r * gh[:, 2 * R:3 * R])
        hn = (1.0 - z) * n + z * h
        H_s[pl.ds(o8, 4), :] = hn
        return hn

    h = lax.fori_loop(0, 64, step, h_s[...])
    h_s[...] = h

    o = jnp.maximum(
        jnp.dot(H_s[...], fc1t[...], preferred_element_type=f32)
        + fc1b[...], 0.0)
    out[0] = jnp.dot(o, fc2t[...], preferred_element_type=f32) + fc2b[...]


def _run_rnn2(x, mrows, p):
    f32 = jnp.float32
    xs3 = jnp.pad(x.T.reshape(64, 64, 4), ((0, 0), (0, 0), (0, 4)),
                  constant_values=-1).reshape(64, 512, 1)
    wih = p['rnn2_w_ih']
    args = (
        xs3,
        p['audio_emb'],
        wih[:, :256].T, wih[:, 256:].T,
        p['rnn2_b_ih'].reshape(1, 1536),
        mrows,
        p['rnn2_w_hh'].T,
        p['rnn2_b_hh'].reshape(1, 1536),
        p['fc1_w'].T, p['fc1_b'].reshape(1, 256),
        p['fc2_w'].T, p['fc2_b'].reshape(1, 256),
    )
    const = lambda shape: pl.BlockSpec(shape, lambda c: (0,) * len(shape))
    in_specs = [
        pl.BlockSpec((1, 512, 1), lambda c: (c, 0, 0)),
        const((256, 256)),
        const((256, 1536)), const((256, 1536)),
        const((1, 1536)),
        const((512, 256)),
        const((512, 1536)),
        const((1, 1536)),
        const((512, 256)), const((1, 256)),
        const((256, 256)), const((1, 256)),
    ]
    out = pl.pallas_call(
        _rnn2_body,
        grid=(64,),
        in_specs=in_specs,
        out_specs=pl.BlockSpec((1, 512, 256), lambda c: (c, 0, 0)),
        out_shape=jax.ShapeDtypeStruct((64, 512, 256), f32),
        scratch_shapes=[
            pltpu.VMEM((256, 1536), f32),
            pltpu.VMEM((512, 1536), f32),
            pltpu.VMEM((512, 1536), f32),
            pltpu.VMEM((512, 512), f32),
            pltpu.VMEM((4, 512), f32),
        ],
    )(*args)
    return out.reshape(64, 64, 8, 256)[:, :, :4, :].transpose(
        2, 0, 1, 3).reshape(4, 4096, 256)


def kernel(x, mels, speakers, params):
    p = params
    q, loss, perp = _run_enc_vq(mels, p)

    q3 = q.transpose(0, 2, 1)
    qrep = jnp.repeat(q3, 2, axis=1)
    spk = p['spk_emb'][speakers]
    spk_b = jnp.broadcast_to(spk[:, None, :], (4, 64, 64))
    m0 = jnp.concatenate([qrep, spk_b], axis=-1)
    m0 = m0.transpose(1, 0, 2)

    y1 = _run_rnn1(m0, p)
    mrows = jnp.pad(y1, ((0, 0), (0, 4), (0, 0))).reshape(512, 256)

    logits = _run_rnn2(x, mrows, p)
    return logits, loss[0, 0], perp[0, 0]


# speedup vs baseline: 6.1759x; 1.0358x over previous
import jax
import jax.numpy as jnp
from jax import lax
from jax.experimental import pallas as pl
from jax.experimental.pallas import tpu as pltpu

_EPS = 1e-5


def _shift_mat(t_in, t_out, off):
    u = lax.broadcasted_iota(jnp.int32, (t_in, t_out), 0)
    t = lax.broadcasted_iota(jnp.int32, (t_in, t_out), 1)
    return (u == t + off).astype(jnp.float32)


def _shift_mat_s2(t_in, t_out, off):
    u = lax.broadcasted_iota(jnp.int32, (t_in, t_out), 0)
    t = lax.broadcasted_iota(jnp.int32, (t_in, t_out), 1)
    return (u == 2 * t + off).astype(jnp.float32)


def _enc_vq_body(mels, w1, w2, w3, w4, w5, w6, b6,
                 g1, be1, g2, be2, g3, be3, g4, be4, g5, be5,
                 cb, cbt,
                 q_out, loss_out, perp_out,
                 sa, sb, sc, sd):
    f32 = jnp.float32

    def conv(src_ref, w_ref, b, sels):
        acc = None
        for k, S in enumerate(sels):
            m = jnp.dot(w_ref[k], src_ref[b], preferred_element_type=f32)
            p = m if S is None else jnp.dot(m, S, preferred_element_type=f32)
            acc = p if acc is None else acc + p
        return acc

    def bn_relu_inplace(s_ref, n_t, g_ref, be_ref):
        n = 4 * n_t
        tot = s_ref[0] + s_ref[1] + s_ref[2] + s_ref[3]
        mean = jnp.sum(tot, axis=1, keepdims=True) / n
        sq = s_ref[0] ** 2 + s_ref[1] ** 2 + s_ref[2] ** 2 + s_ref[3] ** 2
        var = jnp.sum(sq, axis=1, keepdims=True) / n - mean ** 2
        scale = g_ref[...] / jnp.sqrt(var + _EPS)
        shift = be_ref[...] - scale * mean
        for b in range(4):
            s_ref[b] = jnp.maximum(s_ref[b] * scale + shift, 0.0)

    sels1 = [_shift_mat(66, 64, k) for k in range(3)]
    for b in range(4):
        sa[b] = conv(mels, w1, b, sels1)
    bn_relu_inplace(sa, 64, g1, be1)

    sels2 = [_shift_mat(64, 64, -1), None, _shift_mat(64, 64, 1)]
    for b in range(4):
        sb[b] = conv(sa, w2, b, sels2)
    bn_relu_inplace(sb, 64, g2, be2)

    sels3 = [_shift_mat_s2(64, 32, k - 1) for k in range(4)]
    for b in range(4):
        sc[b] = conv(sb, w3, b, sels3)
    bn_relu_inplace(sc, 32, g3, be3)

    sels4 = [_shift_mat(32, 32, -1), None, _shift_mat(32, 32, 1)]
    for b in range(4):
        sd[b] = conv(sc, w4, b, sels4)
    bn_relu_inplace(sd, 32, g4, be4)

    for b in range(4):
        sc[b] = conv(sd, w5, b, sels4)
    bn_relu_inplace(sc, 32, g5, be5)

    e2 = jnp.sum(cb[...] ** 2, axis=1, keepdims=True)
    iota_codes = lax.broadcasted_iota(jnp.int32, (512, 32), 0)
    big = jnp.full((512, 32), 1 << 20, jnp.int32)

    counts = jnp.zeros((512, 1), f32)
    z2_sum = 0.0
    zq_sum = 0.0
    q2_sum = 0.0
    for b in range(4):
        h6 = jnp.dot(w6[...], sc[b],
                     preferred_element_type=f32) + b6[...]
        s = jnp.dot(cb[...], h6, preferred_element_type=f32)
        z2 = jnp.sum(h6 ** 2, axis=0, keepdims=True)
        dist = e2 + z2 - 2.0 * s
        mn = jnp.min(dist, axis=0, keepdims=True)
        cand = jnp.where(dist == mn, iota_codes, big)
        idx = jnp.min(cand, axis=0, keepdims=True)
        oneh = (iota_codes == idx).astype(f32)
        q_out[b] = jnp.dot(cbt[...], oneh, preferred_element_type=f32)
        counts = counts + jnp.sum(oneh, axis=1, keepdims=True)
        z2_sum = z2_sum + jnp.sum(z2)
        zq_sum = zq_sum + jnp.sum(oneh * s)
        q2_sum = q2_sum + jnp.sum(oneh * e2)

    e_latent = (z2_sum - 2.0 * zq_sum + q2_sum) / (128.0 * 64.0)
    loss_out[...] = jnp.full((1, 1), 0.25 * e_latent, f32)
    p = counts / 128.0
    perp_out[...] = jnp.full(
        (1, 1), jnp.exp(-jnp.sum(p * jnp.log(p + 1e-10))), f32)


def _run_enc_vq(mels, p):
    f32 = jnp.float32
    outs = (
        jax.ShapeDtypeStruct((4, 64, 32), f32),
        jax.ShapeDtypeStruct((1, 1), f32),
        jax.ShapeDtypeStruct((1, 1), f32),
    )
    scratch = [
        pltpu.VMEM((4, 512, 64), f32),
        pltpu.VMEM((4, 512, 64), f32),
        pltpu.VMEM((4, 512, 32), f32),
        pltpu.VMEM((4, 512, 32), f32),
    ]
    taps = lambda w: jnp.moveaxis(w, 2, 0)
    args = (
        mels,
        taps(p['enc_w1']), taps(p['enc_w2']), taps(p['enc_w3']),
        taps(p['enc_w4']), taps(p['enc_w5']),
        p['enc_w6'][:, :, 0], p['enc_b6'].reshape(64, 1),
        p['bn1_g'].reshape(512, 1), p['bn1_b'].reshape(512, 1),
        p['bn2_g'].reshape(512, 1), p['bn2_b'].reshape(512, 1),
        p['bn3_g'].reshape(512, 1), p['bn3_b'].reshape(512, 1),
        p['bn4_g'].reshape(512, 1), p['bn4_b'].reshape(512, 1),
        p['bn5_g'].reshape(512, 1), p['bn5_b'].reshape(512, 1),
        p['codebook'], p['codebook'].T,
    )
    return pl.pallas_call(
        _enc_vq_body,
        out_shape=outs,
        scratch_shapes=scratch,
    )(*args)


def _rnn1_body(m0,
               wi0f, wh0f, bi0f, bh0f, wi0r, wh0r, bi0r, bh0r,
               wi1f, wh1f, bi1f, bh1f, wi1r, wh1r, bi1r, bh1r,
               out, y0):
    H = 128

    def gru(in_ref, wi, wh, bi, bh, out_ref, off, reverse):
        def step(i, h):
            t = 63 - i if reverse else i
            xt = in_ref[t]
            gi = jnp.dot(xt, wi[...], preferred_element_type=jnp.float32) \
                + bi[...]
            gh = jnp.dot(h, wh[...], preferred_element_type=jnp.float32) \
                + bh[...]
            r = jax.nn.sigmoid(gi[:, 0:H] + gh[:, 0:H])
            z = jax.nn.sigmoid(gi[:, H:2 * H] + gh[:, H:2 * H])
            n = jnp.tanh(gi[:, 2 * H:3 * H] + r * gh[:, 2 * H:3 * H])
            hn = (1.0 - z) * n + z * h
            out_ref[t, :, off:off + H] = hn
            return hn
        lax.fori_loop(0, 64, step, jnp.zeros((4, H), jnp.float32))

    gru(m0, wi0f, wh0f, bi0f, bh0f, y0, 0, False)
    gru(m0, wi0r, wh0r, bi0r, bh0r, y0, H, True)
    gru(y0, wi1f, wh1f, bi1f, bh1f, out, 0, False)
    gru(y0, wi1r, wh1r, bi1r, bh1r, out, H, True)


def _run_rnn1(m0, p):
    f32 = jnp.float32
    args = [m0]
    for l in range(2):
        for d in ['f', 'r']:
            args.append(p['rnn1_l%d_%s_w_ih' % (l, d)].T)
            args.append(p['rnn1_l%d_%s_w_hh' % (l, d)].T)
            args.append(p['rnn1_l%d_%s_b_ih' % (l, d)].reshape(1, 384))
            args.append(p['rnn1_l%d_%s_b_hh' % (l, d)].reshape(1, 384))
    return pl.pallas_call(
        _rnn1_body,
        out_shape=jax.ShapeDtypeStruct((64, 4, 256), f32),
        scratch_shapes=[pltpu.VMEM((64, 4, 256), f32)],
    )(*args)


def _rnn2_body(xs, aemb, wxt, wmt, bih, mrows, whht, bhh,
               fc1t, fc1b, fc2t, fc2b,
               out, U_s, V_s, G_s, H_s, h_s):
    f32 = jnp.float32
    c = pl.program_id(0)

    bf16 = jnp.bfloat16

    @pl.when(c == 0)
    def _init():
        U_s[...] = jnp.dot(aemb[...], wxt[...],
                           preferred_element_type=f32).astype(bf16)
        V_s[...] = jnp.dot(mrows[...], wmt[...],
                           preferred_element_type=f32) + bih[...]
        h_s[...] = jnp.zeros((4, 512), f32)

    x_blk = xs[0]
    iota_v = lax.broadcasted_iota(jnp.int32, (256, 256), 1)
    oneh = (x_blk == iota_v).astype(bf16)
    G_s[...] = jnp.dot(oneh, U_s[...], preferred_element_type=f32)
    Vc = V_s[pl.ds(pl.multiple_of(c * 8, 8), 4), :]

    R = 512
    whh16 = whht[...]

    def gru_step(gi, h):
        gh = jnp.dot(h.astype(bf16), whh16,
                     preferred_element_type=f32) + bhh[...]
        r = jax.nn.sigmoid(gi[:, 0:R] + gh[:, 0:R])
        z = jax.nn.sigmoid(gi[:, R:2 * R] + gh[:, R:2 * R])
        n = jnp.tanh(gi[:, 2 * R:3 * R] + r * gh[:, 2 * R:3 * R])
        return (1.0 - z) * n + z * h

    def pair(i, h):
        o8 = pl.multiple_of(i * 8, 8)
        g8 = G_s[pl.ds(o8, 8), :]
        ha = gru_step(g8[0:4, :] + Vc, h)
        hb = gru_step(g8[4:8, :] + Vc, ha)
        H_s[pl.ds(o8, 8), :] = jnp.concatenate([ha, hb], axis=0)
        return hb

    h = lax.fori_loop(0, 32, pair, h_s[...])
    h_s[...] = h

    o = jnp.maximum(
        jnp.dot(H_s[...], fc1t[...], preferred_element_type=f32)
        + fc1b[...], 0.0)
    out[0] = jnp.dot(o, fc2t[...], preferred_element_type=f32) + fc2b[...]


def _run_rnn2(x, mrows, p):
    f32 = jnp.float32
    xs3 = x.T.reshape(64, 256, 1)
    wih = p['rnn2_w_ih']
    args = (
        xs3,
        p['audio_emb'],
        wih[:, :256].T, wih[:, 256:].T,
        p['rnn2_b_ih'].reshape(1, 1536),
        mrows,
        p['rnn2_w_hh'].T.astype(jnp.bfloat16),
        p['rnn2_b_hh'].reshape(1, 1536),
        p['fc1_w'].T, p['fc1_b'].reshape(1, 256),
        p['fc2_w'].T, p['fc2_b'].reshape(1, 256),
    )
    const = lambda shape: pl.BlockSpec(shape, lambda c: (0,) * len(shape))
    in_specs = [
        pl.BlockSpec((1, 256, 1), lambda c: (c, 0, 0)),
        const((256, 256)),
        const((256, 1536)), const((256, 1536)),
        const((1, 1536)),
        const((512, 256)),
        const((512, 1536)),
        const((1, 1536)),
        const((512, 256)), const((1, 256)),
        const((256, 256)), const((1, 256)),
    ]
    out = pl.pallas_call(
        _rnn2_body,
        grid=(64,),
        in_specs=in_specs,
        out_specs=pl.BlockSpec((1, 256, 256), lambda c: (c, 0, 0)),
        out_shape=jax.ShapeDtypeStruct((64, 256, 256), f32),
        scratch_shapes=[
            pltpu.VMEM((256, 1536), jnp.bfloat16),
            pltpu.VMEM((512, 1536), f32),
            pltpu.VMEM((256, 1536), f32),
            pltpu.VMEM((256, 512), f32),
            pltpu.VMEM((4, 512), f32),
        ],
    )(*args)
    return out.reshape(64, 64, 4, 256).transpose(2, 0, 1, 3).reshape(
        4, 4096, 256)


def kernel(x, mels, speakers, params):
    p = params
    q, loss, perp = _run_enc_vq(mels, p)

    q3 = q.transpose(0, 2, 1)
    qrep = jnp.repeat(q3, 2, axis=1)
    spk = p['spk_emb'][speakers]
    spk_b = jnp.broadcast_to(spk[:, None, :], (4, 64, 64))
    m0 = jnp.concatenate([qrep, spk_b], axis=-1)
    m0 = m0.transpose(1, 0, 2)

    y1 = _run_rnn1(m0, p)
    mrows = jnp.pad(y1, ((0, 0), (0, 4), (0, 0))).reshape(512, 256)

    logits = _run_rnn2(x, mrows, p)
    return logits, loss[0, 0], perp[0, 0]


# speedup vs baseline: 6.3479x; 1.0278x over previous
import functools

import jax
import jax.numpy as jnp
from jax import lax
from jax.experimental import pallas as pl
from jax.experimental.pallas import tpu as pltpu
from jax.experimental.pallas import tpu_sc as plsc

_EPS = 1e-5


def _spk_sc_body(table_hbm, idx_hbm, out_hbm, idx_v, rows_v, sem):
    first = (lax.axis_index("s") == 0) & (lax.axis_index("c") == 0)

    @pl.when(first)
    def _():
        pltpu.sync_copy(idx_hbm, idx_v)
        pltpu.async_copy(table_hbm.at[idx_v], rows_v, sem).wait()
        pltpu.sync_copy(rows_v, out_hbm)


def _run_spk_gather(speakers, table):
    idx8 = jnp.pad(speakers, (0, 4))
    t128 = jnp.pad(table, ((0, 0), (0, 128 - table.shape[1])))
    k = functools.partial(
        pl.kernel,
        mesh=plsc.VectorSubcoreMesh(core_axis_name="c", subcore_axis_name="s"),
        out_type=jax.ShapeDtypeStruct((8, 128), jnp.float32),
        scratch_types=[
            pltpu.VMEM((8,), jnp.int32),
            pltpu.VMEM((8, 128), jnp.float32),
            pltpu.SemaphoreType.DMA,
        ],
    )(_spk_sc_body)
    return k(t128, idx8)[:4, :table.shape[1]]


def _shift_mat(t_in, t_out, off):
    u = lax.broadcasted_iota(jnp.int32, (t_in, t_out), 0)
    t = lax.broadcasted_iota(jnp.int32, (t_in, t_out), 1)
    return (u == t + off).astype(jnp.float32)


def _shift_mat_s2(t_in, t_out, off):
    u = lax.broadcasted_iota(jnp.int32, (t_in, t_out), 0)
    t = lax.broadcasted_iota(jnp.int32, (t_in, t_out), 1)
    return (u == 2 * t + off).astype(jnp.float32)


def _enc_vq_body(mels, w1, w2, w3, w4, w5, w6, b6,
                 g1, be1, g2, be2, g3, be3, g4, be4, g5, be5,
                 cb, cbt,
                 q_out, loss_out, perp_out,
                 sa, sb, sc, sd):
    f32 = jnp.float32

    def conv(src_ref, w_ref, b, sels):
        acc = None
        for k, S in enumerate(sels):
            m = jnp.dot(w_ref[k], src_ref[b], preferred_element_type=f32)
            p = m if S is None else jnp.dot(m, S, preferred_element_type=f32)
            acc = p if acc is None else acc + p
        return acc

    def bn_relu_inplace(s_ref, n_t, g_ref, be_ref):
        n = 4 * n_t
        tot = s_ref[0] + s_ref[1] + s_ref[2] + s_ref[3]
        mean = jnp.sum(tot, axis=1, keepdims=True) / n
        sq = s_ref[0] ** 2 + s_ref[1] ** 2 + s_ref[2] ** 2 + s_ref[3] ** 2
        var = jnp.sum(sq, axis=1, keepdims=True) / n - mean ** 2
        scale = g_ref[...] / jnp.sqrt(var + _EPS)
        shift = be_ref[...] - scale * mean
        for b in range(4):
            s_ref[b] = jnp.maximum(s_ref[b] * scale + shift, 0.0)

    sels1 = [_shift_mat(66, 64, k) for k in range(3)]
    for b in range(4):
        sa[b] = conv(mels, w1, b, sels1)
    bn_relu_inplace(sa, 64, g1, be1)

    sels2 = [_shift_mat(64, 64, -1), None, _shift_mat(64, 64, 1)]
    for b in range(4):
        sb[b] = conv(sa, w2, b, sels2)
    bn_relu_inplace(sb, 64, g2, be2)

    sels3 = [_shift_mat_s2(64, 32, k - 1) for k in range(4)]
    for b in range(4):
        sc[b] = conv(sb, w3, b, sels3)
    bn_relu_inplace(sc, 32, g3, be3)

    sels4 = [_shift_mat(32, 32, -1), None, _shift_mat(32, 32, 1)]
    for b in range(4):
        sd[b] = conv(sc, w4, b, sels4)
    bn_relu_inplace(sd, 32, g4, be4)

    for b in range(4):
        sc[b] = conv(sd, w5, b, sels4)
    bn_relu_inplace(sc, 32, g5, be5)

    e2 = jnp.sum(cb[...] ** 2, axis=1, keepdims=True)
    iota_codes = lax.broadcasted_iota(jnp.int32, (512, 32), 0)
    big = jnp.full((512, 32), 1 << 20, jnp.int32)

    counts = jnp.zeros((512, 1), f32)
    z2_sum = 0.0
    zq_sum = 0.0
    q2_sum = 0.0
    for b in range(4):
        h6 = jnp.dot(w6[...], sc[b],
                     preferred_element_type=f32) + b6[...]
        s = jnp.dot(cb[...], h6, preferred_element_type=f32)
        z2 = jnp.sum(h6 ** 2, axis=0, keepdims=True)
        dist = e2 + z2 - 2.0 * s
        mn = jnp.min(dist, axis=0, keepdims=True)
        cand = jnp.where(dist == mn, iota_codes, big)
        idx = jnp.min(cand, axis=0, keepdims=True)
        oneh = (iota_codes == idx).astype(f32)
        q_out[b] = jnp.dot(cbt[...], oneh, preferred_element_type=f32)
        counts = counts + jnp.sum(oneh, axis=1, keepdims=True)
        z2_sum = z2_sum + jnp.sum(z2)
        zq_sum = zq_sum + jnp.sum(oneh * s)
        q2_sum = q2_sum + jnp.sum(oneh * e2)

    e_latent = (z2_sum - 2.0 * zq_sum + q2_sum) / (128.0 * 64.0)
    loss_out[...] = jnp.full((1, 1), 0.25 * e_latent, f32)
    p = counts / 128.0
    perp_out[...] = jnp.full(
        (1, 1), jnp.exp(-jnp.sum(p * jnp.log(p + 1e-10))), f32)


def _run_enc_vq(mels, p):
    f32 = jnp.float32
    outs = (
        jax.ShapeDtypeStruct((4, 64, 32), f32),
        jax.ShapeDtypeStruct((1, 1), f32),
        jax.ShapeDtypeStruct((1, 1), f32),
    )
    scratch = [
        pltpu.VMEM((4, 512, 64), f32),
        pltpu.VMEM((4, 512, 64), f32),
        pltpu.VMEM((4, 512, 32), f32),
        pltpu.VMEM((4, 512, 32), f32),
    ]
    taps = lambda w: jnp.moveaxis(w, 2, 0)
    args = (
        mels,
        taps(p['enc_w1']), taps(p['enc_w2']), taps(p['enc_w3']),
        taps(p['enc_w4']), taps(p['enc_w5']),
        p['enc_w6'][:, :, 0], p['enc_b6'].reshape(64, 1),
        p['bn1_g'].reshape(512, 1), p['bn1_b'].reshape(512, 1),
        p['bn2_g'].reshape(512, 1), p['bn2_b'].reshape(512, 1),
        p['bn3_g'].reshape(512, 1), p['bn3_b'].reshape(512, 1),
        p['bn4_g'].reshape(512, 1), p['bn4_b'].reshape(512, 1),
        p['bn5_g'].reshape(512, 1), p['bn5_b'].reshape(512, 1),
        p['codebook'], p['codebook'].T,
    )
    return pl.pallas_call(
        _enc_vq_body,
        out_shape=outs,
        scratch_shapes=scratch,
    )(*args)


def _rnn1_body(m0,
               wi0f, wh0f, bi0f, bh0f, wi0r, wh0r, bi0r, bh0r,
               wi1f, wh1f, bi1f, bh1f, wi1r, wh1r, bi1r, bh1r,
               out, y0):
    H = 128

    def gru(in_ref, wi, wh, bi, bh, out_ref, off, reverse):
        def step(i, h):
            t = 63 - i if reverse else i
            xt = in_ref[t]
            gi = jnp.dot(xt, wi[...], preferred_element_type=jnp.float32) \
                + bi[...]
            gh = jnp.dot(h, wh[...], preferred_element_type=jnp.float32) \
                + bh[...]
            r = jax.nn.sigmoid(gi[:, 0:H] + gh[:, 0:H])
            z = jax.nn.sigmoid(gi[:, H:2 * H] + gh[:, H:2 * H])
            n = jnp.tanh(gi[:, 2 * H:3 * H] + r * gh[:, 2 * H:3 * H])
            hn = (1.0 - z) * n + z * h
            out_ref[t, :, off:off + H] = hn
            return hn
        lax.fori_loop(0, 64, step, jnp.zeros((4, H), jnp.float32))

    gru(m0, wi0f, wh0f, bi0f, bh0f, y0, 0, False)
    gru(m0, wi0r, wh0r, bi0r, bh0r, y0, H, True)
    gru(y0, wi1f, wh1f, bi1f, bh1f, out, 0, False)
    gru(y0, wi1r, wh1r, bi1r, bh1r, out, H, True)


def _run_rnn1(m0, p):
    f32 = jnp.float32
    args = [m0]
    for l in range(2):
        for d in ['f', 'r']:
            args.append(p['rnn1_l%d_%s_w_ih' % (l, d)].T)
            args.append(p['rnn1_l%d_%s_w_hh' % (l, d)].T)
            args.append(p['rnn1_l%d_%s_b_ih' % (l, d)].reshape(1, 384))
            args.append(p['rnn1_l%d_%s_b_hh' % (l, d)].reshape(1, 384))
    return pl.pallas_call(
        _rnn1_body,
        out_shape=jax.ShapeDtypeStruct((64, 4, 256), f32),
        scratch_shapes=[pltpu.VMEM((64, 4, 256), f32)],
    )(*args)


def _rnn2_body(xs, aemb, wxt, wmt, bih, mrows, whr, whz, whn, bhh,
               fc1t, fc1b, fc2t, fc2b,
               out, U_s, V_s, G_s, H_s, h_s):
    f32 = jnp.float32
    c = pl.program_id(0)

    bf16 = jnp.bfloat16

    @pl.when(c == 0)
    def _init():
        U_s[...] = jnp.dot(aemb[...], wxt[...],
                           preferred_element_type=f32).astype(bf16)
        V_s[...] = jnp.dot(mrows[...], wmt[...],
                           preferred_element_type=f32) + bih[...]
        h_s[...] = jnp.zeros((4, 512), f32)

    x_blk = xs[0]
    iota_v = lax.broadcasted_iota(jnp.int32, (256, 256), 1)
    oneh = (x_blk == iota_v).astype(bf16)
    G_s[...] = jnp.dot(oneh, U_s[...], preferred_element_type=f32)
    Vc = V_s[pl.ds(pl.multiple_of(c * 8, 8), 4), :]

    R = 512

    def gru_step(gi, h):
        h16 = h.astype(bf16)
        ghr = jnp.dot(h16, whr[...], preferred_element_type=f32) \
            + bhh[:, 0:R]
        ghz = jnp.dot(h16, whz[...], preferred_element_type=f32) \
            + bhh[:, R:2 * R]
        ghn = jnp.dot(h16, whn[...], preferred_element_type=f32) \
            + bhh[:, 2 * R:3 * R]
        r = jax.nn.sigmoid(gi[:, 0:R] + ghr)
        z = jax.nn.sigmoid(gi[:, R:2 * R] + ghz)
        n = jnp.tanh(gi[:, 2 * R:3 * R] + r * ghn)
        return (1.0 - z) * n + z * h

    def pair(i, h):
        o8 = pl.multiple_of(i * 8, 8)
        g8 = G_s[pl.ds(o8, 8), :]
        ha = gru_step(g8[0:4, :] + Vc, h)
        hb = gru_step(g8[4:8, :] + Vc, ha)
        H_s[pl.ds(o8, 8), :] = jnp.concatenate([ha, hb], axis=0)
        return hb

    h = lax.fori_loop(0, 32, pair, h_s[...])
    h_s[...] = h

    o = jnp.maximum(
        jnp.dot(H_s[...], fc1t[...], preferred_element_type=f32)
        + fc1b[...], 0.0)
    out[0] = jnp.dot(o, fc2t[...], preferred_element_type=f32) + fc2b[...]


def _run_rnn2(x, mrows, p):
    f32 = jnp.float32
    xs3 = x.T.reshape(64, 256, 1)
    wih = p['rnn2_w_ih']
    args = (
        xs3,
        p['audio_emb'],
        wih[:, :256].T, wih[:, 256:].T,
        p['rnn2_b_ih'].reshape(1, 1536),
        mrows,
        p['rnn2_w_hh'][0:512].T.astype(jnp.bfloat16),
        p['rnn2_w_hh'][512:1024].T.astype(jnp.bfloat16),
        p['rnn2_w_hh'][1024:1536].T.astype(jnp.bfloat16),
        p['rnn2_b_hh'].reshape(1, 1536),
        p['fc1_w'].T, p['fc1_b'].reshape(1, 256),
        p['fc2_w'].T, p['fc2_b'].reshape(1, 256),
    )
    const = lambda shape: pl.BlockSpec(shape, lambda c: (0,) * len(shape))
    in_specs = [
        pl.BlockSpec((1, 256, 1), lambda c: (c, 0, 0)),
        const((256, 256)),
        const((256, 1536)), const((256, 1536)),
        const((1, 1536)),
        const((512, 256)),
        const((512, 512)), const((512, 512)), const((512, 512)),
        const((1, 1536)),
        const((512, 256)), const((1, 256)),
        const((256, 256)), const((1, 256)),
    ]
    out = pl.pallas_call(
        _rnn2_body,
        grid=(64,),
        in_specs=in_specs,
        out_specs=pl.BlockSpec((1, 256, 256), lambda c: (c, 0, 0)),
        out_shape=jax.ShapeDtypeStruct((64, 256, 256), f32),
        scratch_shapes=[
            pltpu.VMEM((256, 1536), jnp.bfloat16),
            pltpu.VMEM((512, 1536), f32),
            pltpu.VMEM((256, 1536), f32),
            pltpu.VMEM((256, 512), f32),
            pltpu.VMEM((4, 512), f32),
        ],
    )(*args)
    return out.reshape(64, 64, 4, 256).transpose(2, 0, 1, 3).reshape(
        4, 4096, 256)


def kernel(x, mels, speakers, params):
    p = params
    q, loss, perp = _run_enc_vq(mels, p)

    q3 = q.transpose(0, 2, 1)
    qrep = jnp.repeat(q3, 2, axis=1)
    spk = _run_spk_gather(speakers, p['spk_emb'])
    spk_b = jnp.broadcast_to(spk[:, None, :], (4, 64, 64))
    m0 = jnp.concatenate([qrep, spk_b], axis=-1)
    m0 = m0.transpose(1, 0, 2)

    y1 = _run_rnn1(m0, p)
    mrows = jnp.pad(y1, ((0, 0), (0, 4), (0, 0))).reshape(512, 256)

    logits = _run_rnn2(x, mrows, p)
    return logits, loss[0, 0], perp[0, 0]


# speedup vs baseline: 6.4046x; 1.0089x over previous
import functools

import jax
import jax.numpy as jnp
from jax import lax
from jax.experimental import pallas as pl
from jax.experimental.pallas import tpu as pltpu
from jax.experimental.pallas import tpu_sc as plsc

_EPS = 1e-5


def _spk_sc_body(table_hbm, idx_hbm, out_hbm, idx_v, rows_v, sem):
    first = (lax.axis_index("s") == 0) & (lax.axis_index("c") == 0)

    @pl.when(first)
    def _():
        pltpu.sync_copy(idx_hbm, idx_v)
        pltpu.async_copy(table_hbm.at[idx_v], rows_v, sem).wait()
        pltpu.sync_copy(rows_v, out_hbm)


def _run_spk_gather(speakers, table):
    idx8 = jnp.pad(speakers, (0, 4))
    t128 = jnp.pad(table, ((0, 0), (0, 128 - table.shape[1])))
    k = functools.partial(
        pl.kernel,
        mesh=plsc.VectorSubcoreMesh(core_axis_name="c", subcore_axis_name="s"),
        out_type=jax.ShapeDtypeStruct((8, 128), jnp.float32),
        scratch_types=[
            pltpu.VMEM((8,), jnp.int32),
            pltpu.VMEM((8, 128), jnp.float32),
            pltpu.SemaphoreType.DMA,
        ],
    )(_spk_sc_body)
    return k(t128, idx8)[:4, :table.shape[1]]


def _shift_mat(t_in, t_out, off):
    u = lax.broadcasted_iota(jnp.int32, (t_in, t_out), 0)
    t = lax.broadcasted_iota(jnp.int32, (t_in, t_out), 1)
    return (u == t + off).astype(jnp.float32)


def _shift_mat_s2(t_in, t_out, off):
    u = lax.broadcasted_iota(jnp.int32, (t_in, t_out), 0)
    t = lax.broadcasted_iota(jnp.int32, (t_in, t_out), 1)
    return (u == 2 * t + off).astype(jnp.float32)


def _enc_vq_body(mels, w1, w2, w3, w4, w5, w6, b6,
                 g1, be1, g2, be2, g3, be3, g4, be4, g5, be5,
                 cb, cbt,
                 q_out, loss_out, perp_out,
                 sa, sb, sc, sd):
    f32 = jnp.float32

    def conv(src_ref, w_ref, b, sels):
        acc = None
        for k, S in enumerate(sels):
            m = jnp.dot(w_ref[k], src_ref[b], preferred_element_type=f32)
            p = m if S is None else jnp.dot(m, S, preferred_element_type=f32)
            acc = p if acc is None else acc + p
        return acc

    def bn_relu_inplace(s_ref, n_t, g_ref, be_ref):
        n = 4 * n_t
        tot = s_ref[0] + s_ref[1] + s_ref[2] + s_ref[3]
        mean = jnp.sum(tot, axis=1, keepdims=True) / n
        sq = s_ref[0] ** 2 + s_ref[1] ** 2 + s_ref[2] ** 2 + s_ref[3] ** 2
        var = jnp.sum(sq, axis=1, keepdims=True) / n - mean ** 2
        scale = g_ref[...] / jnp.sqrt(var + _EPS)
        shift = be_ref[...] - scale * mean
        for b in range(4):
            s_ref[b] = jnp.maximum(s_ref[b] * scale + shift, 0.0)

    sels1 = [_shift_mat(66, 64, k) for k in range(3)]
    for b in range(4):
        sa[b] = conv(mels, w1, b, sels1)
    bn_relu_inplace(sa, 64, g1, be1)

    sels2 = [_shift_mat(64, 64, -1), None, _shift_mat(64, 64, 1)]
    for b in range(4):
        sb[b] = conv(sa, w2, b, sels2)
    bn_relu_inplace(sb, 64, g2, be2)

    sels3 = [_shift_mat_s2(64, 32, k - 1) for k in range(4)]
    for b in range(4):
        sc[b] = conv(sb, w3, b, sels3)
    bn_relu_inplace(sc, 32, g3, be3)

    sels4 = [_shift_mat(32, 32, -1), None, _shift_mat(32, 32, 1)]
    for b in range(4):
        sd[b] = conv(sc, w4, b, sels4)
    bn_relu_inplace(sd, 32, g4, be4)

    for b in range(4):
        sc[b] = conv(sd, w5, b, sels4)
    bn_relu_inplace(sc, 32, g5, be5)

    e2 = jnp.sum(cb[...] ** 2, axis=1, keepdims=True)
    iota_codes = lax.broadcasted_iota(jnp.int32, (512, 32), 0)
    big = jnp.full((512, 32), 1 << 20, jnp.int32)

    counts = jnp.zeros((512, 1), f32)
    z2_sum = 0.0
    zq_sum = 0.0
    q2_sum = 0.0
    for b in range(4):
        h6 = jnp.dot(w6[...], sc[b],
                     preferred_element_type=f32) + b6[...]
        s = jnp.dot(cb[...], h6, preferred_element_type=f32)
        z2 = jnp.sum(h6 ** 2, axis=0, keepdims=True)
        dist = e2 + z2 - 2.0 * s
        mn = jnp.min(dist, axis=0, keepdims=True)
        cand = jnp.where(dist == mn, iota_codes, big)
        idx = jnp.min(cand, axis=0, keepdims=True)
        oneh = (iota_codes == idx).astype(f32)
        q_out[b] = jnp.dot(cbt[...], oneh, preferred_element_type=f32)
        counts = counts + jnp.sum(oneh, axis=1, keepdims=True)
        z2_sum = z2_sum + jnp.sum(z2)
        zq_sum = zq_sum + jnp.sum(oneh * s)
        q2_sum = q2_sum + jnp.sum(oneh * e2)

    e_latent = (z2_sum - 2.0 * zq_sum + q2_sum) / (128.0 * 64.0)
    loss_out[...] = jnp.full((1, 1), 0.25 * e_latent, f32)
    p = counts / 128.0
    perp_out[...] = jnp.full(
        (1, 1), jnp.exp(-jnp.sum(p * jnp.log(p + 1e-10))), f32)


def _run_enc_vq(mels, p):
    f32 = jnp.float32
    outs = (
        jax.ShapeDtypeStruct((4, 64, 32), f32),
        jax.ShapeDtypeStruct((1, 1), f32),
        jax.ShapeDtypeStruct((1, 1), f32),
    )
    scratch = [
        pltpu.VMEM((4, 512, 64), f32),
        pltpu.VMEM((4, 512, 64), f32),
        pltpu.VMEM((4, 512, 32), f32),
        pltpu.VMEM((4, 512, 32), f32),
    ]
    taps = lambda w: jnp.moveaxis(w, 2, 0)
    args = (
        mels,
        taps(p['enc_w1']), taps(p['enc_w2']), taps(p['enc_w3']),
        taps(p['enc_w4']), taps(p['enc_w5']),
        p['enc_w6'][:, :, 0], p['enc_b6'].reshape(64, 1),
        p['bn1_g'].reshape(512, 1), p['bn1_b'].reshape(512, 1),
        p['bn2_g'].reshape(512, 1), p['bn2_b'].reshape(512, 1),
        p['bn3_g'].reshape(512, 1), p['bn3_b'].reshape(512, 1),
        p['bn4_g'].reshape(512, 1), p['bn4_b'].reshape(512, 1),
        p['bn5_g'].reshape(512, 1), p['bn5_b'].reshape(512, 1),
        p['codebook'], p['codebook'].T,
    )
    return pl.pallas_call(
        _enc_vq_body,
        out_shape=outs,
        scratch_shapes=scratch,
    )(*args)


def _rnn1_body(m0,
               wi0f, wh0f, bi0f, bh0f, wi0r, wh0r, bi0r, bh0r,
               wi1f, wh1f, bi1f, bh1f, wi1r, wh1r, bi1r, bh1r,
               out, y0):
    H = 128
    f32 = jnp.float32

    def cell(xt, h, wi, wh, bi, bh):
        gi = jnp.dot(xt, wi[...], preferred_element_type=f32) + bi[...]
        gh = jnp.dot(h, wh[...], preferred_element_type=f32) + bh[...]
        r = jax.nn.sigmoid(gi[:, 0:H] + gh[:, 0:H])
        z = jax.nn.sigmoid(gi[:, H:2 * H] + gh[:, H:2 * H])
        n = jnp.tanh(gi[:, 2 * H:3 * H] + r * gh[:, 2 * H:3 * H])
        return (1.0 - z) * n + z * h

    def bigru(in_ref, wif, whf, bif, bhf, wir, whr, bir, bhr, out_ref):
        def step(i, carry):
            hf, hr = carry
            tr = 63 - i
            hf = cell(in_ref[i], hf, wif, whf, bif, bhf)
            hr = cell(in_ref[tr], hr, wir, whr, bir, bhr)
            out_ref[i, :, 0:H] = hf
            out_ref[tr, :, H:2 * H] = hr
            return (hf, hr)
        z4 = jnp.zeros((4, H), f32)
        lax.fori_loop(0, 64, step, (z4, z4))

    bigru(m0, wi0f, wh0f, bi0f, bh0f, wi0r, wh0r, bi0r, bh0r, y0)
    bigru(y0, wi1f, wh1f, bi1f, bh1f, wi1r, wh1r, bi1r, bh1r, out)


def _run_rnn1(m0, p):
    f32 = jnp.float32
    args = [m0]
    for l in range(2):
        for d in ['f', 'r']:
            args.append(p['rnn1_l%d_%s_w_ih' % (l, d)].T)
            args.append(p['rnn1_l%d_%s_w_hh' % (l, d)].T)
            args.append(p['rnn1_l%d_%s_b_ih' % (l, d)].reshape(1, 384))
            args.append(p['rnn1_l%d_%s_b_hh' % (l, d)].reshape(1, 384))
    return pl.pallas_call(
        _rnn1_body,
        out_shape=jax.ShapeDtypeStruct((64, 4, 256), f32),
        scratch_shapes=[pltpu.VMEM((64, 4, 256), f32)],
    )(*args)


def _rnn2_body(xs, aemb, wxt, wmt, bih, mrows, whr, whz, whn, bhh,
               fc1t, fc1b, fc2t, fc2b,
               out, U_s, V_s, G_s, H_s, h_s):
    f32 = jnp.float32
    c = pl.program_id(0)

    @pl.when(c == 0)
    def _init():
        U_s[...] = jnp.dot(aemb[...], wxt[...], preferred_element_type=f32)
        V_s[...] = jnp.dot(mrows[...], wmt[...],
                           preferred_element_type=f32) + bih[...]
        h_s[...] = jnp.zeros((4, 512), f32)

    x_blk = xs[0]
    iota_v = lax.broadcasted_iota(jnp.int32, (256, 256), 1)
    oneh = (x_blk == iota_v).astype(f32)
    G_s[...] = jnp.dot(oneh, U_s[...], preferred_element_type=f32)
    Vc = V_s[pl.ds(pl.multiple_of(c * 8, 8), 4), :]

    R = 512

    def gru_step(gi, h):
        ghr = jnp.dot(h, whr[...], preferred_element_type=f32) \
            + bhh[:, 0:R]
        ghz = jnp.dot(h, whz[...], preferred_element_type=f32) \
            + bhh[:, R:2 * R]
        ghn = jnp.dot(h, whn[...], preferred_element_type=f32) \
            + bhh[:, 2 * R:3 * R]
        r = jax.nn.sigmoid(gi[:, 0:R] + ghr)
        z = jax.nn.sigmoid(gi[:, R:2 * R] + ghz)
        n = jnp.tanh(gi[:, 2 * R:3 * R] + r * ghn)
        return (1.0 - z) * n + z * h

    def pair(i, h):
        o8 = pl.multiple_of(i * 8, 8)
        g8 = G_s[pl.ds(o8, 8), :]
        ha = gru_step(g8[0:4, :] + Vc, h)
        hb = gru_step(g8[4:8, :] + Vc, ha)
        H_s[pl.ds(o8, 8), :] = jnp.concatenate([ha, hb], axis=0)
        return hb

    h = lax.fori_loop(0, 32, pair, h_s[...])
    h_s[...] = h

    o = jnp.maximum(
        jnp.dot(H_s[...], fc1t[...], preferred_element_type=f32)
        + fc1b[...], 0.0)
    out[0] = jnp.dot(o, fc2t[...], preferred_element_type=f32) + fc2b[...]


def _run_rnn2(x, mrows, p):
    f32 = jnp.float32
    xs3 = x.T.reshape(64, 256, 1)
    wih = p['rnn2_w_ih']
    args = (
        xs3,
        p['audio_emb'],
        wih[:, :256].T, wih[:, 256:].T,
        p['rnn2_b_ih'].reshape(1, 1536),
        mrows,
        p['rnn2_w_hh'][0:512].T,
        p['rnn2_w_hh'][512:1024].T,
        p['rnn2_w_hh'][1024:1536].T,
        p['rnn2_b_hh'].reshape(1, 1536),
        p['fc1_w'].T, p['fc1_b'].reshape(1, 256),
        p['fc2_w'].T, p['fc2_b'].reshape(1, 256),
    )
    const = lambda shape: pl.BlockSpec(shape, lambda c: (0,) * len(shape))
    in_specs = [
        pl.BlockSpec((1, 256, 1), lambda c: (c, 0, 0)),
        const((256, 256)),
        const((256, 1536)), const((256, 1536)),
        const((1, 1536)),
        const((512, 256)),
        const((512, 512)), const((512, 512)), const((512, 512)),
        const((1, 1536)),
        const((512, 256)), const((1, 256)),
        const((256, 256)), const((1, 256)),
    ]
    out = pl.pallas_call(
        _rnn2_body,
        grid=(64,),
        in_specs=in_specs,
        out_specs=pl.BlockSpec((1, 256, 256), lambda c: (c, 0, 0)),
        out_shape=jax.ShapeDtypeStruct((64, 256, 256), f32),
        scratch_shapes=[
            pltpu.VMEM((256, 1536), f32),
            pltpu.VMEM((512, 1536), f32),
            pltpu.VMEM((256, 1536), f32),
            pltpu.VMEM((256, 512), f32),
            pltpu.VMEM((4, 512), f32),
        ],
    )(*args)
    return out.reshape(64, 64, 4, 256).transpose(2, 0, 1, 3).reshape(
        4, 4096, 256)


def kernel(x, mels, speakers, params):
    p = params
    q, loss, perp = _run_enc_vq(mels, p)

    q3 = q.transpose(0, 2, 1)
    qrep = jnp.repeat(q3, 2, axis=1)
    spk = _run_spk_gather(speakers, p['spk_emb'])
    spk_b = jnp.broadcast_to(spk[:, None, :], (4, 64, 64))
    m0 = jnp.concatenate([qrep, spk_b], axis=-1)
    m0 = m0.transpose(1, 0, 2)

    y1 = _run_rnn1(m0, p)
    mrows = jnp.pad(y1, ((0, 0), (0, 4), (0, 0))).reshape(512, 256)

    logits = _run_rnn2(x, mrows, p)
    return logits, loss[0, 0], perp[0, 0]
